# baseline (device time: 6626 ns/iter reference)
import jax
import jax.numpy as jnp
from jax import lax
from jax.experimental import pallas as pl
from jax.experimental.pallas import tpu as pltpu

N_CHUNKS = 2


def kernel(x, dy, gamma):
    del gamma
    m, d = x.shape
    rows = m // N_CHUNKS

    def body(
        x_hbm,
        dy_hbm,
        out_ref,
        xbuf,
        dybuf,
        xsems,
        ysems,
        comm_ref,
        send_sem,
        recv_sem,
        out_sem,
    ):
        my_x = lax.axis_index("x")
        my_y = lax.axis_index("y")
        my_z = lax.axis_index("z")
        partner = (1 - my_x, my_y, my_z)

        barrier_sem = pltpu.get_barrier_semaphore()
        pl.semaphore_signal(
            barrier_sem,
            inc=1,
            device_id=partner,
            device_id_type=pl.DeviceIdType.MESH,
        )

        copies = []
        for c in range(N_CHUNKS):
            sl = pl.ds(c * rows, rows)
            cp_x = pltpu.make_async_copy(x_hbm.at[sl, :], xbuf.at[sl, :], xsems.at[c])
            cp_y = pltpu.make_async_copy(dy_hbm.at[sl, :], dybuf.at[sl, :], ysems.at[c])
            cp_x.start()
            cp_y.start()
            copies.append((cp_x, cp_y))

        ones = jnp.ones((d, 128), jnp.float32)
        inv_d = 1.0 / d
        dg = jnp.zeros((1, d), jnp.float32)
        db = jnp.zeros((1, d), jnp.float32)
        for c in range(N_CHUNKS):
            copies[c][0].wait()
            copies[c][1].wait()
            sl = pl.ds(c * rows, rows)
            xv = xbuf[sl, :]
            dyv = dybuf[sl, :]
            sums = jnp.dot(jnp.concatenate([xv, xv * xv], axis=0), ones)
            mu = sums[0:rows, 0:1] * inv_d
            ex2 = sums[rows : 2 * rows, 0:1] * inv_d
            var = ex2 - mu * mu
            rstd = lax.rsqrt(var + 1e-5)
            xhat = (xv - mu) * rstd
            dg = dg + jnp.sum(dyv * xhat, axis=0, keepdims=True)
            db = db + jnp.sum(dyv, axis=0, keepdims=True)

        comm_ref[0, 0:1, :] = dg
        comm_ref[0, 1:2, :] = db

        pl.semaphore_wait(barrier_sem, 1)

        rdma = pltpu.make_async_remote_copy(
            src_ref=comm_ref.at[0],
            dst_ref=comm_ref.at[1],
            send_sem=send_sem,
            recv_sem=recv_sem,
            device_id=partner,
            device_id_type=pl.DeviceIdType.MESH,
        )
        rdma.start()
        rdma.wait()

        comm_ref[0, :, :] = comm_ref[0] + comm_ref[1]
        out_cp = pltpu.make_async_copy(comm_ref.at[0], out_ref, out_sem)
        out_cp.start()
        out_cp.wait()

    return pl.pallas_call(
        body,
        out_shape=jax.ShapeDtypeStruct((2, d), jnp.float32),
        in_specs=[
            pl.BlockSpec(memory_space=pl.ANY),
            pl.BlockSpec(memory_space=pl.ANY),
        ],
        out_specs=pl.BlockSpec(memory_space=pl.ANY),
        scratch_shapes=[
            pltpu.VMEM((m, d), jnp.float32),
            pltpu.VMEM((m, d), jnp.float32),
            pltpu.SemaphoreType.DMA((N_CHUNKS,)),
            pltpu.SemaphoreType.DMA((N_CHUNKS,)),
            pltpu.VMEM((2, 2, d), jnp.float32),
            pltpu.SemaphoreType.DMA,
            pltpu.SemaphoreType.DMA,
            pltpu.SemaphoreType.DMA,
        ],
        compiler_params=pltpu.CompilerParams(collective_id=0),
    )(
        pltpu.with_memory_space_constraint(x, pltpu.MemorySpace.HBM),
        pltpu.with_memory_space_constraint(dy, pltpu.MemorySpace.HBM),
    )


# device time: 6016 ns/iter; 1.1014x vs baseline; 1.1014x over previous
import jax
import jax.numpy as jnp
from jax import lax
from jax.experimental import pallas as pl
from jax.experimental.pallas import tpu as pltpu

N_CHUNKS = 2


def kernel(x, dy, gamma):
    del gamma
    m, d = x.shape
    rows = m // N_CHUNKS

    def body(
        x_hbm,
        dy_hbm,
        out_ref,
        xbuf,
        dybuf,
        xsems,
        ysems,
        comm_ref,
        send_sem,
        recv_sem,
        out_sem,
    ):
        my_x = lax.axis_index("x")
        my_y = lax.axis_index("y")
        my_z = lax.axis_index("z")
        partner = (1 - my_x, my_y, my_z)

        barrier_sem = pltpu.get_barrier_semaphore()
        pl.semaphore_signal(
            barrier_sem,
            inc=1,
            device_id=partner,
            device_id_type=pl.DeviceIdType.MESH,
        )

        copies = []
        for c in range(N_CHUNKS):
            sl = pl.ds(c * rows, rows)
            cp_x = pltpu.make_async_copy(x_hbm.at[sl, :], xbuf.at[sl, :], xsems.at[c])
            cp_y = pltpu.make_async_copy(dy_hbm.at[sl, :], dybuf.at[sl, :], ysems.at[c])
            cp_x.start()
            cp_y.start()
            copies.append((cp_x, cp_y))

        dg = jnp.zeros((1, d), jnp.float32)
        db = jnp.zeros((1, d), jnp.float32)
        for c in range(N_CHUNKS):
            copies[c][0].wait()
            copies[c][1].wait()
            sl = pl.ds(c * rows, rows)
            xv = xbuf[sl, :]
            dyv = dybuf[sl, :]
            mu = jnp.mean(xv, axis=1, keepdims=True)
            var = jnp.mean((xv - mu) * (xv - mu), axis=1, keepdims=True)
            rstd = lax.rsqrt(var + 1e-5)
            xhat = (xv - mu) * rstd
            dg = dg + jnp.sum(dyv * xhat, axis=0, keepdims=True)
            db = db + jnp.sum(dyv, axis=0, keepdims=True)

        comm_ref[0, 0:1, :] = dg
        comm_ref[0, 1:2, :] = db

        pl.semaphore_wait(barrier_sem, 1)

        rdma = pltpu.make_async_remote_copy(
            src_ref=comm_ref.at[0],
            dst_ref=comm_ref.at[1],
            send_sem=send_sem,
            recv_sem=recv_sem,
            device_id=partner,
            device_id_type=pl.DeviceIdType.MESH,
        )
        rdma.start()
        rdma.wait()

        comm_ref[0, :, :] = comm_ref[0] + comm_ref[1]
        out_cp = pltpu.make_async_copy(comm_ref.at[0], out_ref, out_sem)
        out_cp.start()
        out_cp.wait()

    return pl.pallas_call(
        body,
        out_shape=jax.ShapeDtypeStruct((2, d), jnp.float32),
        in_specs=[
            pl.BlockSpec(memory_space=pl.ANY),
            pl.BlockSpec(memory_space=pl.ANY),
        ],
        out_specs=pl.BlockSpec(memory_space=pl.ANY),
        scratch_shapes=[
            pltpu.VMEM((m, d), jnp.float32),
            pltpu.VMEM((m, d), jnp.float32),
            pltpu.SemaphoreType.DMA((N_CHUNKS,)),
            pltpu.SemaphoreType.DMA((N_CHUNKS,)),
            pltpu.VMEM((2, 2, d), jnp.float32),
            pltpu.SemaphoreType.DMA,
            pltpu.SemaphoreType.DMA,
            pltpu.SemaphoreType.DMA,
        ],
        compiler_params=pltpu.CompilerParams(collective_id=0),
    )(
        pltpu.with_memory_space_constraint(x, pltpu.MemorySpace.HBM),
        pltpu.with_memory_space_constraint(dy, pltpu.MemorySpace.HBM),
    )


# device time: 5949 ns/iter; 1.1138x vs baseline; 1.0113x over previous
import jax
import jax.numpy as jnp
from jax import lax
from jax.experimental import pallas as pl
from jax.experimental.pallas import tpu as pltpu

N_CHUNKS = 2


def kernel(x, dy, gamma):
    del gamma
    m, d = x.shape
    rows = m // N_CHUNKS

    def body(
        x_hbm,
        dy_hbm,
        out_ref,
        xbuf,
        dybuf,
        xsems,
        ysems,
        comm_ref,
        send_sem,
        recv_sem,
        out_sem,
    ):
        my_x = lax.axis_index("x")
        my_y = lax.axis_index("y")
        my_z = lax.axis_index("z")
        partner = (1 - my_x, my_y, my_z)

        barrier_sem = pltpu.get_barrier_semaphore()
        pl.semaphore_signal(
            barrier_sem,
            inc=1,
            device_id=partner,
            device_id_type=pl.DeviceIdType.MESH,
        )

        copies = []
        for c in range(N_CHUNKS):
            sl = pl.ds(c * rows, rows)
            cp_x = pltpu.make_async_copy(x_hbm.at[sl, :], xbuf.at[sl, :], xsems.at[c])
            cp_y = pltpu.make_async_copy(dy_hbm.at[sl, :], dybuf.at[sl, :], ysems.at[c])
            cp_x.start()
            cp_y.start()
            copies.append((cp_x, cp_y))

        dg = jnp.zeros((1, d), jnp.float32)
        db = jnp.zeros((1, d), jnp.float32)
        for c in range(N_CHUNKS):
            copies[c][0].wait()
            copies[c][1].wait()
            sl = pl.ds(c * rows, rows)
            xv = xbuf[sl, :]
            dyv = dybuf[sl, :]
            mu = jnp.mean(xv, axis=1, keepdims=True)
            ex2 = jnp.mean(xv * xv, axis=1, keepdims=True)
            var = ex2 - mu * mu
            rstd = lax.rsqrt(var + 1e-5)
            dg = dg + jnp.sum(dyv * (xv * rstd - mu * rstd), axis=0, keepdims=True)
            db = db + jnp.sum(dyv, axis=0, keepdims=True)

        comm_ref[0, 0:1, :] = dg
        comm_ref[0, 1:2, :] = db

        pl.semaphore_wait(barrier_sem, 1)

        rdma = pltpu.make_async_remote_copy(
            src_ref=comm_ref.at[0],
            dst_ref=comm_ref.at[1],
            send_sem=send_sem,
            recv_sem=recv_sem,
            device_id=partner,
            device_id_type=pl.DeviceIdType.MESH,
        )
        rdma.start()
        rdma.wait()

        comm_ref[0, :, :] = comm_ref[0] + comm_ref[1]
        out_cp = pltpu.make_async_copy(comm_ref.at[0], out_ref, out_sem)
        out_cp.start()
        out_cp.wait()

    return pl.pallas_call(
        body,
        out_shape=jax.ShapeDtypeStruct((2, d), jnp.float32),
        in_specs=[
            pl.BlockSpec(memory_space=pl.ANY),
            pl.BlockSpec(memory_space=pl.ANY),
        ],
        out_specs=pl.BlockSpec(memory_space=pl.ANY),
        scratch_shapes=[
            pltpu.VMEM((m, d), jnp.float32),
            pltpu.VMEM((m, d), jnp.float32),
            pltpu.SemaphoreType.DMA((N_CHUNKS,)),
            pltpu.SemaphoreType.DMA((N_CHUNKS,)),
            pltpu.VMEM((2, 2, d), jnp.float32),
            pltpu.SemaphoreType.DMA,
            pltpu.SemaphoreType.DMA,
            pltpu.SemaphoreType.DMA,
        ],
        compiler_params=pltpu.CompilerParams(collective_id=0),
    )(
        pltpu.with_memory_space_constraint(x, pltpu.MemorySpace.HBM),
        pltpu.with_memory_space_constraint(dy, pltpu.MemorySpace.HBM),
    )
